# baseline (speedup 1.0000x reference)
import jax
import jax.numpy as jnp
from jax import lax
from jax.experimental import pallas as pl
from jax.experimental.pallas import tpu as pltpu

_CONTRACT0 = (((0,), (0,)), ((), ()))


def _make_ms_cam_kernel(NB):
    def _ms_cam_kernel(x_ref, w1_ref, b1_ref, w2_ref, b2_ref,
                       g1_ref, gb1_ref, g2_ref, gb2_ref, o_ref,
                       w1s, b1s, w2s, b2s, g1s, gb1s, g2s, gb2s, sems):
        @pl.when(pl.program_id(0) == 0)
        def _load_weights():
            for i, (src, dst) in enumerate([
                    (w1_ref, w1s), (b1_ref, b1s), (w2_ref, w2s),
                    (b2_ref, b2s), (g1_ref, g1s), (gb1_ref, gb1s),
                    (g2_ref, g2s), (gb2_ref, gb2s)]):
                pltpu.make_async_copy(src, dst, sems.at[i]).start()
            for i, (src, dst) in enumerate([
                    (w1_ref, w1s), (b1_ref, b1s), (w2_ref, w2s),
                    (b2_ref, b2s), (g1_ref, g1s), (gb1_ref, gb1s),
                    (g2_ref, g2s), (gb2_ref, gb2s)]):
                pltpu.make_async_copy(src, dst, sems.at[i]).wait()

        bsum = b2s[...] + gb2s[...]
        for i in range(NB):
            x = x_ref[i]
            C, HW = x.shape

            m = jnp.sum(x, axis=1, keepdims=True) * (1.0 / HW)
            mb = jnp.broadcast_to(m, (C, 128))
            hg = jnp.maximum(
                jnp.dot(g1s[...], mb, preferred_element_type=jnp.float32)
                + gb1s[...], 0.0)
            xg = jnp.dot(g2s[...], hg,
                         preferred_element_type=jnp.float32)[:, 0:1]

            h = jnp.maximum(
                jnp.dot(w1s[...], x, preferred_element_type=jnp.float32)
                + b1s[...], 0.0)
            xl = jnp.dot(w2s[...], h, preferred_element_type=jnp.float32)

            gate = 0.5 + 0.5 * jnp.tanh((xl + (xg + bsum)) * 0.5)
            o_ref[i] = (x * gate).astype(o_ref.dtype)
    return _ms_cam_kernel


def kernel(x_nchw, w1, b1, w2, b2, g1, gb1, g2, gb2):
    N, C, H, W = x_nchw.shape
    HW = H * W
    Ci = w1.shape[1]

    x = x_nchw.reshape(N, C, HW)
    w1t, w2t, g1t, g2t = w1.T, w2.T, g1.T, g2.T
    b1c = b1.reshape(Ci, 1)
    b2c = b2.reshape(C, 1)
    gb1c = gb1.reshape(Ci, 1)
    gb2c = gb2.reshape(C, 1)

    NB = 2 if N % 2 == 0 else 1
    anyspec = pl.BlockSpec(memory_space=pl.ANY)
    out = pl.pallas_call(
        _make_ms_cam_kernel(NB),
        out_shape=jax.ShapeDtypeStruct((N, C, HW), x.dtype),
        grid=(N // NB,),
        in_specs=[pl.BlockSpec((NB, C, HW), lambda n: (n, 0, 0))]
                 + [anyspec] * 8,
        out_specs=pl.BlockSpec((NB, C, HW), lambda n: (n, 0, 0)),
        scratch_shapes=[
            pltpu.VMEM((Ci, C), jnp.float32),
            pltpu.VMEM((Ci, 1), jnp.float32),
            pltpu.VMEM((C, Ci), jnp.float32),
            pltpu.VMEM((C, 1), jnp.float32),
            pltpu.VMEM((Ci, C), jnp.float32),
            pltpu.VMEM((Ci, 1), jnp.float32),
            pltpu.VMEM((C, Ci), jnp.float32),
            pltpu.VMEM((C, 1), jnp.float32),
            pltpu.SemaphoreType.DMA((8,)),
        ],
        compiler_params=pltpu.CompilerParams(
            dimension_semantics=("arbitrary",)),
    )(x, w1t, b1c, w2t, b2c, g1t, gb1c, g2t, gb2c)

    return out.reshape(N, C, H, W)


# speedup vs baseline: 1.0454x; 1.0454x over previous
import jax
import jax.numpy as jnp
from jax.experimental import pallas as pl
from jax.experimental.pallas import tpu as pltpu


def _make_ms_cam_kernel(NB):
    def _ms_cam_kernel(x_ref, w1_ref, b1_ref, w2_ref, b2_ref,
                       g1_ref, gb1_ref, g2_ref, gb2_ref, o_ref,
                       w1r, w2r, g1r, g2r,
                       w1s, b1s, w2s, bss, g1s, gb1s, g2s, gb2s, sems):
        @pl.when(pl.program_id(0) == 0)
        def _load_weights():
            copies = [(w1_ref, w1r), (w2_ref, w2r), (g1_ref, g1r),
                      (g2_ref, g2r), (b1_ref, b1s), (gb1_ref, gb1s),
                      (b2_ref, bss), (gb2_ref, gb2s)]
            for i, (src, dst) in enumerate(copies):
                pltpu.make_async_copy(src, dst, sems.at[i]).start()
            for i, (src, dst) in enumerate(copies):
                pltpu.make_async_copy(src, dst, sems.at[i]).wait()
            w1s[...] = w1r[...].T
            w2s[...] = w2r[...].T
            g1s[...] = g1r[...].T
            g2s[...] = g2r[...].T
            bss[...] = bss[...] + gb2s[...]

        for i in range(NB):
            x = x_ref[i]
            C, HW = x.shape

            m = jnp.sum(x, axis=1, keepdims=True) * (1.0 / HW)
            mb = jnp.broadcast_to(m, (C, 128))
            hg = jnp.maximum(
                jnp.dot(g1s[...], mb, preferred_element_type=jnp.float32)
                + gb1s[...], 0.0)
            xg = jnp.dot(g2s[...], hg,
                         preferred_element_type=jnp.float32)[:, 0:1]

            h = jnp.maximum(
                jnp.dot(w1s[...], x, preferred_element_type=jnp.float32)
                + b1s[...], 0.0)
            xl = jnp.dot(w2s[...], h, preferred_element_type=jnp.float32)

            gate = 0.5 + 0.5 * jnp.tanh((xl + (xg + bss[...])) * 0.5)
            o_ref[i] = (x * gate).astype(o_ref.dtype)
    return _ms_cam_kernel


def kernel(x_nchw, w1, b1, w2, b2, g1, gb1, g2, gb2):
    N, C, H, W = x_nchw.shape
    HW = H * W
    Ci = w1.shape[1]

    x = x_nchw.reshape(N, C, HW)
    b1c = b1.reshape(Ci, 1)
    b2c = b2.reshape(C, 1)
    gb1c = gb1.reshape(Ci, 1)
    gb2c = gb2.reshape(C, 1)

    NB = 2 if N % 2 == 0 else 1
    anyspec = pl.BlockSpec(memory_space=pl.ANY)
    out = pl.pallas_call(
        _make_ms_cam_kernel(NB),
        out_shape=jax.ShapeDtypeStruct((N, C, HW), x.dtype),
        grid=(N // NB,),
        in_specs=[pl.BlockSpec((NB, C, HW), lambda n: (n, 0, 0))]
                 + [anyspec] * 8,
        out_specs=pl.BlockSpec((NB, C, HW), lambda n: (n, 0, 0)),
        scratch_shapes=[
            pltpu.VMEM((C, Ci), jnp.float32),
            pltpu.VMEM((Ci, C), jnp.float32),
            pltpu.VMEM((C, Ci), jnp.float32),
            pltpu.VMEM((Ci, C), jnp.float32),
            pltpu.VMEM((Ci, C), jnp.float32),
            pltpu.VMEM((Ci, 1), jnp.float32),
            pltpu.VMEM((C, Ci), jnp.float32),
            pltpu.VMEM((C, 1), jnp.float32),
            pltpu.VMEM((Ci, C), jnp.float32),
            pltpu.VMEM((Ci, 1), jnp.float32),
            pltpu.VMEM((C, Ci), jnp.float32),
            pltpu.VMEM((C, 1), jnp.float32),
            pltpu.SemaphoreType.DMA((8,)),
        ],
        compiler_params=pltpu.CompilerParams(
            dimension_semantics=("arbitrary",)),
    )(x, w1, b1c, w2, b2c, g1, gb1c, g2, gb2c)

    return out.reshape(N, C, H, W)
